# baseline (speedup 1.0000x reference)
import jax
import jax.numpy as jnp
from jax.experimental import pallas as pl

N = 50000
E = 800000
SKIP = 16


def _trilinear_sample(vol, coords):
    C, Dd, Hh, Ww = vol.shape
    x = (coords[:, 0] + 1.0) * 0.5 * (Ww - 1)
    y = (coords[:, 1] + 1.0) * 0.5 * (Hh - 1)
    z = (coords[:, 2] + 1.0) * 0.5 * (Dd - 1)
    x0 = jnp.clip(jnp.floor(x), 0, Ww - 1)
    y0 = jnp.clip(jnp.floor(y), 0, Hh - 1)
    z0 = jnp.clip(jnp.floor(z), 0, Dd - 1)
    x1 = jnp.clip(x0 + 1, 0, Ww - 1)
    y1 = jnp.clip(y0 + 1, 0, Hh - 1)
    z1 = jnp.clip(z0 + 1, 0, Dd - 1)
    wx = x - x0
    wy = y - y0
    wz = z - z0
    x0i = x0.astype(jnp.int32); x1i = x1.astype(jnp.int32)
    y0i = y0.astype(jnp.int32); y1i = y1.astype(jnp.int32)
    z0i = z0.astype(jnp.int32); z1i = z1.astype(jnp.int32)
    flat = vol.reshape(C, -1)
    def g(zi, yi, xi):
        idx = (zi * Hh + yi) * Ww + xi
        return jnp.take(flat, idx, axis=1)
    c000 = g(z0i, y0i, x0i); c001 = g(z0i, y0i, x1i)
    c010 = g(z0i, y1i, x0i); c011 = g(z0i, y1i, x1i)
    c100 = g(z1i, y0i, x0i); c101 = g(z1i, y0i, x1i)
    c110 = g(z1i, y1i, x0i); c111 = g(z1i, y1i, x1i)
    c00 = c000 * (1 - wx) + c001 * wx
    c01 = c010 * (1 - wx) + c011 * wx
    c10 = c100 * (1 - wx) + c101 * wx
    c11 = c110 * (1 - wx) + c111 * wx
    c0 = c00 * (1 - wy) + c01 * wy
    c1 = c10 * (1 - wy) + c11 * wy
    return c0 * (1 - wz) + c1 * wz


def _graph_conv(x, src, dst, deg, Ws, Wn, b):
    agg = jax.ops.segment_sum(x[src], dst, num_segments=N)
    agg = agg / deg[:, None]
    return x @ Ws + agg @ Wn + b


def _final_add(v_ref, d_ref, o_ref):
    o_ref[...] = v_ref[...] + d_ref[...]


def kernel(vertices, voxel_decoder_features, edge_index, W_skip, b_skip, Ws0, Wn0, b0, Ws1, Wn1, b1, Ws2, Wn2, b2, Wv0s, Wv0n, bv0, Wv1s, Wv1n, bv1, Wv2s, Wv2n, bv2):
    v = vertices[0]
    vol = voxel_decoder_features[0, :SKIP]
    sampled = _trilinear_sample(vol, v).T
    skipped = sampled @ W_skip + b_skip
    feats = jnp.concatenate([skipped, v], axis=1)
    src = edge_index[0]
    dst = edge_index[1]
    deg = jnp.clip(jax.ops.segment_sum(jnp.ones(E, dtype=jnp.float32), dst, num_segments=N), 1.0, None)
    h = jax.nn.relu(_graph_conv(feats, src, dst, deg, Ws0, Wn0, b0))
    h = jax.nn.relu(_graph_conv(h, src, dst, deg, Ws1, Wn1, b1))
    latent = _graph_conv(h, src, dst, deg, Ws2, Wn2, b2)
    g1 = jax.nn.relu(_graph_conv(latent, src, dst, deg, Wv0s, Wv0n, bv0))
    g2 = jax.nn.relu(_graph_conv(g1, src, dst, deg, Wv1s, Wv1n, bv1))
    deltaV = _graph_conv(g2, src, dst, deg, Wv2s, Wv2n, bv2)
    new_v = pl.pallas_call(
        _final_add,
        grid=(50,),
        in_specs=[pl.BlockSpec((1000, 3), lambda i: (i, 0)),
                  pl.BlockSpec((1000, 3), lambda i: (i, 0))],
        out_specs=pl.BlockSpec((1000, 3), lambda i: (i, 0)),
        out_shape=jax.ShapeDtypeStruct(v.shape, v.dtype),
    )(v, deltaV)
    return (new_v[None], latent[None])


# speedup vs baseline: 5.2147x; 5.2147x over previous
import functools

import jax
import jax.numpy as jnp
from jax import lax
from jax.experimental import pallas as pl
from jax.experimental.pallas import tpu as pltpu
from jax.experimental.pallas import tpu_sc as plsc

N = 50000
E = 800000
SKIP = 16
LAT = 64
G = 32

NP_ = 50176
BLK = 1024
NPB = NP_ // BLK
WORKERS = 32
PER_W = NP_ // WORKERS
PER_T = NP_ // 16
E_PAD = 819200
EC = E_PAD // WORKERS
CH = 512
NCH = EC // CH

_mesh = plsc.VectorSubcoreMesh(core_axis_name="c", subcore_axis_name="s",
                               num_cores=2, num_subcores=16)


def _corner_gather_body(vol_ref, idx_ref, out_ref, idxv, rows, sem):
    c = lax.axis_index("c")
    s = lax.axis_index("s")
    wid = s * 2 + c
    base = wid * PER_W
    for k in range(8):
        pltpu.sync_copy(idx_ref.at[pl.ds(k * NP_ + base, PER_W)], idxv)
        pltpu.async_copy(vol_ref.at[idxv], rows, sem).wait()
        pltpu.sync_copy(rows, out_ref.at[pl.ds(k * NP_ + base, PER_W)])


_corner_gather = pl.kernel(
    _corner_gather_body,
    out_type=jax.ShapeDtypeStruct((8 * NP_, SKIP), jnp.float32),
    mesh=_mesh,
    compiler_params=pltpu.CompilerParams(use_tc_tiling_on_sc=False),
    scratch_types=[
        pltpu.VMEM((PER_W,), jnp.int32),
        pltpu.VMEM((PER_W, SKIP), jnp.float32),
        pltpu.SemaphoreType.DMA,
    ],
)


def _segsum_body(ntab, src_dst_zero_tabs_outs_scratch):
    refs = src_dst_zero_tabs_outs_scratch
    src_ref, dst_ref, zero_ref = refs[0], refs[1], refs[2]
    tabs = refs[3:3 + ntab]
    outs = refs[3 + ntab:3 + 2 * ntab]
    sv, dv, rows, sem, acc = refs[3 + 2 * ntab:]
    c = lax.axis_index("c")
    s = lax.axis_index("s")
    wid = s * 2 + c
    ebase = wid * EC
    rbase = s * PER_T
    obase = c * NP_ + s * PER_T
    for t in range(ntab):
        pltpu.sync_copy(zero_ref.at[pl.ds(rbase, PER_T)],
                        acc.at[pl.ds(rbase, PER_T)])
        plsc.subcore_barrier()

        def chunk(i, carry, _t=t):
            off = ebase + i * CH
            pltpu.sync_copy(src_ref.at[pl.ds(off, CH)], sv)
            pltpu.sync_copy(dst_ref.at[pl.ds(off, CH)], dv)
            pltpu.async_copy(tabs[_t].at[sv], rows, sem).wait()
            pltpu.sync_copy(rows, acc.at[dv], add=True)
            return carry

        lax.fori_loop(0, NCH, chunk, 0)
        plsc.subcore_barrier()
        pltpu.sync_copy(acc.at[pl.ds(rbase, PER_T)],
                        outs[t].at[pl.ds(obase, PER_T)])


def _make_segsum(ntab, d):
    def body(*refs):
        _segsum_body(ntab, refs)

    return pl.kernel(
        body,
        out_type=[jax.ShapeDtypeStruct((2 * NP_, d), jnp.float32)] * ntab,
        mesh=_mesh,
        compiler_params=pltpu.CompilerParams(use_tc_tiling_on_sc=False),
        scratch_types=[
            pltpu.VMEM((CH,), jnp.int32),
            pltpu.VMEM((CH,), jnp.int32),
            pltpu.VMEM((CH, d), jnp.float32),
            pltpu.SemaphoreType.DMA,
            pltpu.VMEM_SHARED((NP_, d), jnp.float32),
        ],
    )


_segsum32x1 = _make_segsum(1, 32)
_segsum32x2 = _make_segsum(2, 32)
_segsum16x1 = _make_segsum(1, 16)


def _row_spec(k):
    return pl.BlockSpec((BLK, k), lambda i: (i, 0))


def _p_specs(d):
    return [pl.BlockSpec((BLK, d), lambda i: (i, 0)),
            pl.BlockSpec((BLK, d), lambda i: (i + NPB, 0))]


def _full(a):
    return pl.BlockSpec(a.shape, lambda i: tuple(0 for _ in a.shape))


def _tri_prep_body(vt_ref, idx_ref, w_ref):
    half = 0.5 * (G - 1)
    x = (vt_ref[0:1, :] + 1.0) * half
    y = (vt_ref[1:2, :] + 1.0) * half
    z = (vt_ref[2:3, :] + 1.0) * half
    x0 = jnp.clip(jnp.floor(x), 0.0, G - 1)
    y0 = jnp.clip(jnp.floor(y), 0.0, G - 1)
    z0 = jnp.clip(jnp.floor(z), 0.0, G - 1)
    x1 = jnp.minimum(x0 + 1.0, G - 1)
    y1 = jnp.minimum(y0 + 1.0, G - 1)
    z1 = jnp.minimum(z0 + 1.0, G - 1)
    wx = x - x0
    wy = y - y0
    wz = z - z0
    x0i = x0.astype(jnp.int32); x1i = x1.astype(jnp.int32)
    y0i = y0.astype(jnp.int32); y1i = y1.astype(jnp.int32)
    z0i = z0.astype(jnp.int32); z1i = z1.astype(jnp.int32)

    def ind(zi, yi, xi):
        return (zi * G + yi) * G + xi

    idx_ref[...] = jnp.concatenate([
        ind(z0i, y0i, x0i), ind(z0i, y0i, x1i),
        ind(z0i, y1i, x0i), ind(z0i, y1i, x1i),
        ind(z1i, y0i, x0i), ind(z1i, y0i, x1i),
        ind(z1i, y1i, x0i), ind(z1i, y1i, x1i),
    ], axis=0)
    ax = 1.0 - wx
    ay = 1.0 - wy
    az = 1.0 - wz
    w_ref[...] = jnp.concatenate([
        az * ay * ax, az * ay * wx, az * wy * ax, az * wy * wx,
        wz * ay * ax, wz * ay * wx, wz * wy * ax, wz * wy * wx,
    ], axis=0)


def _tri_prep(vt):
    return pl.pallas_call(
        _tri_prep_body,
        grid=(NPB,),
        in_specs=[pl.BlockSpec((3, BLK), lambda i: (0, i))],
        out_specs=[pl.BlockSpec((8, BLK), lambda i: (0, i)),
                   pl.BlockSpec((8, BLK), lambda i: (0, i))],
        out_shape=[jax.ShapeDtypeStruct((8, NP_), jnp.int32),
                   jax.ShapeDtypeStruct((8, NP_), jnp.float32)],
    )(vt)


def _feats_body(r_ref, w_ref, v_ref, ws_ref, bs_ref, t0_ref):
    rows = r_ref[...]
    w = w_ref[...]
    sampled = jnp.zeros((BLK, SKIP), jnp.float32)
    for k in range(8):
        sampled = sampled + w[:, k:k + 1] * rows[k]
    skipped = jnp.dot(sampled, ws_ref[...],
                      preferred_element_type=jnp.float32) + bs_ref[...]
    ones = jnp.ones((BLK, 1), jnp.float32)
    zeros = jnp.zeros((BLK, 32 - (SKIP + 1) - 3 - 1), jnp.float32)
    t0_ref[...] = jnp.concatenate([skipped, v_ref[...], ones, zeros], axis=1)


def _feats(rows8, w8, vpad, W_skip, b_skip):
    return pl.pallas_call(
        _feats_body,
        grid=(NPB,),
        in_specs=[pl.BlockSpec((8, BLK, SKIP), lambda i: (0, i, 0)),
                  _row_spec(8), _row_spec(3), _full(W_skip), _full(b_skip)],
        out_specs=_row_spec(32),
        out_shape=jax.ShapeDtypeStruct((NP_, 32), jnp.float32),
    )(rows8, w8, vpad, W_skip, b_skip)


def _layer0_body(x_ref, pa_ref, pb_ref, ws_ref, wn_ref, b_ref,
                 ha_ref, hb_ref, dinv_ref):
    p = pa_ref[...] + pb_ref[...]
    deg = jnp.maximum(p[:, 20:21], 1.0)
    dinv = 1.0 / deg
    agg = p[:, :20] * dinv
    x = x_ref[...]
    h = jnp.dot(x[:, :20], ws_ref[...], preferred_element_type=jnp.float32)
    h = h + jnp.dot(agg, wn_ref[...], preferred_element_type=jnp.float32)
    h = jax.nn.relu(h + b_ref[...])
    ha_ref[...] = h[:, :32]
    hb_ref[...] = h[:, 32:]
    dinv_ref[...] = dinv


def _layer0(t0, p0, Ws0, Wn0, b0):
    return pl.pallas_call(
        _layer0_body,
        grid=(NPB,),
        in_specs=[_row_spec(32)] + _p_specs(32)
        + [_full(Ws0), _full(Wn0), _full(b0)],
        out_specs=[_row_spec(32), _row_spec(32), _row_spec(1)],
        out_shape=[jax.ShapeDtypeStruct((NP_, 32), jnp.float32),
                   jax.ShapeDtypeStruct((NP_, 32), jnp.float32),
                   jax.ShapeDtypeStruct((NP_, 1), jnp.float32)],
    )(t0, p0, p0, Ws0, Wn0, b0)


def _mid_body(n_extra, extra_fn, *refs):
    (xa_ref, xb_ref, paa_ref, pab_ref, pba_ref, pbb_ref, dinv_ref,
     wsa_ref, wsb_ref, wna_ref, wnb_ref, b_ref) = refs[:12]
    extra_refs = refs[12:12 + n_extra]
    out_refs = refs[12 + n_extra:]
    dinv = dinv_ref[...]
    agga = (paa_ref[...] + pab_ref[...]) * dinv
    aggb = (pba_ref[...] + pbb_ref[...]) * dinv
    h = jnp.dot(xa_ref[...], wsa_ref[...], preferred_element_type=jnp.float32)
    h = h + jnp.dot(xb_ref[...], wsb_ref[...],
                    preferred_element_type=jnp.float32)
    h = h + jnp.dot(agga, wna_ref[...], preferred_element_type=jnp.float32)
    h = h + jnp.dot(aggb, wnb_ref[...], preferred_element_type=jnp.float32)
    h = h + b_ref[...]
    extra_fn(h, extra_refs, out_refs)


def _relu_split(h, extra_refs, out_refs):
    h = jax.nn.relu(h)
    out_refs[0][...] = h[:, :32]
    out_refs[1][...] = h[:, 32:]


def _mid_layer(extra_fn, out_shapes, xa, xb, pa, pb, dinv,
               wsa, wsb, wna, wnb, b, extra_full=()):
    body = functools.partial(_mid_body, len(extra_full), extra_fn)
    return pl.pallas_call(
        body,
        grid=(NPB,),
        in_specs=[_row_spec(32), _row_spec(32)]
        + _p_specs(32) + _p_specs(32) + [_row_spec(1)]
        + [_full(wsa), _full(wsb), _full(wna), _full(wnb), _full(b)]
        + [_full(a) for a in extra_full],
        out_specs=[_row_spec(s[1]) for s in out_shapes],
        out_shape=[jax.ShapeDtypeStruct(s, jnp.float32) for s in out_shapes],
    )(xa, xb, pa, pa, pb, pb, dinv, wsa, wsb, wna, wnb, b, *extra_full)


def _layer3_body(la_ref, lb_ref, pa_ref, pb_ref, dinv_ref, wsa_ref, wsb_ref,
                 b_ref, wv1n_ref, g1_ref, z4_ref):
    agg = (pa_ref[...] + pb_ref[...]) * dinv_ref[...]
    h = jnp.dot(la_ref[...], wsa_ref[...], preferred_element_type=jnp.float32)
    h = h + jnp.dot(lb_ref[...], wsb_ref[...],
                    preferred_element_type=jnp.float32)
    g1 = jax.nn.relu(h + agg + b_ref[...])
    g1_ref[...] = g1
    z4_ref[...] = jnp.dot(g1, wv1n_ref[...],
                          preferred_element_type=jnp.float32)


def _layer3(la, lb, p3, dinv, wsa, wsb, bv0, Wv1n):
    return pl.pallas_call(
        _layer3_body,
        grid=(NPB,),
        in_specs=[_row_spec(32), _row_spec(32)] + _p_specs(32)
        + [_row_spec(1), _full(wsa), _full(wsb), _full(bv0), _full(Wv1n)],
        out_specs=[_row_spec(32), _row_spec(16)],
        out_shape=[jax.ShapeDtypeStruct((NP_, 32), jnp.float32),
                   jax.ShapeDtypeStruct((NP_, 16), jnp.float32)],
    )(la, lb, p3, p3, dinv, wsa, wsb, bv0, Wv1n)


def _layer4_body(g1_ref, pa_ref, pb_ref, dinv_ref, ws_ref, b_ref, g2_ref):
    agg = (pa_ref[...] + pb_ref[...]) * dinv_ref[...]
    h = jnp.dot(g1_ref[...], ws_ref[...], preferred_element_type=jnp.float32)
    g2_ref[...] = jax.nn.relu(h + agg + b_ref[...])


def _layer4(g1, p4, dinv, Wv1s, bv1):
    return pl.pallas_call(
        _layer4_body,
        grid=(NPB,),
        in_specs=[_row_spec(32)] + _p_specs(16)
        + [_row_spec(1), _full(Wv1s), _full(bv1)],
        out_specs=_row_spec(16),
        out_shape=jax.ShapeDtypeStruct((NP_, 16), jnp.float32),
    )(g1, p4, p4, dinv, Wv1s, bv1)


def _layer5_body(g2_ref, pa_ref, pb_ref, dinv_ref, v_ref, ws_ref, wn_ref,
                 b_ref, out_ref):
    agg = (pa_ref[...] + pb_ref[...]) * dinv_ref[...]
    d = jnp.dot(g2_ref[...], ws_ref[...], preferred_element_type=jnp.float32)
    d = d + jnp.dot(agg, wn_ref[...], preferred_element_type=jnp.float32)
    out_ref[...] = v_ref[...] + d + b_ref[...]


def _layer5(g2, p5, dinv, vpad, Wv2s, Wv2n, bv2):
    return pl.pallas_call(
        _layer5_body,
        grid=(NPB,),
        in_specs=[_row_spec(16)] + _p_specs(16)
        + [_row_spec(1), _row_spec(3), _full(Wv2s), _full(Wv2n), _full(bv2)],
        out_specs=_row_spec(3),
        out_shape=jax.ShapeDtypeStruct((NP_, 3), jnp.float32),
    )(g2, p5, p5, dinv, vpad, Wv2s, Wv2n, bv2)


def kernel(vertices, voxel_decoder_features, edge_index, W_skip, b_skip,
           Ws0, Wn0, b0, Ws1, Wn1, b1, Ws2, Wn2, b2, Wv0s, Wv0n, bv0,
           Wv1s, Wv1n, bv1, Wv2s, Wv2n, bv2):
    v = vertices[0]
    vpad = jnp.pad(v, ((0, NP_ - N), (0, 0)))
    vt = vpad.T
    vol = voxel_decoder_features[0, :SKIP].reshape(SKIP, G * G * G).T
    vol = jnp.asarray(vol, jnp.float32)
    src = jnp.pad(edge_index[0], (0, E_PAD - E))
    dst = jnp.pad(edge_index[1], (0, E_PAD - E), constant_values=N)
    zeros32 = jnp.zeros((NP_, 32), jnp.float32)
    zeros16 = jnp.zeros((NP_, 16), jnp.float32)

    idx8, w8r = _tri_prep(vt)
    rows8 = _corner_gather(vol, idx8.reshape(-1))
    t0 = _feats(rows8.reshape(8, NP_, SKIP), w8r.T, vpad, W_skip,
                b_skip.reshape(1, -1))

    (p0,) = _segsum32x1(src, dst, zeros32, t0)
    h0a, h0b, dinv = _layer0(t0, p0, Ws0[:20], Wn0[:20], b0.reshape(1, -1))

    p1a, p1b = _segsum32x2(src, dst, zeros32, h0a, h0b)
    h1a, h1b = _mid_layer(
        _relu_split, [(NP_, 32), (NP_, 32)], h0a, h0b, p1a, p1b, dinv,
        Ws1[:32], Ws1[32:], Wn1[:32], Wn1[32:], b1.reshape(1, -1))

    p2a, p2b = _segsum32x2(src, dst, zeros32, h1a, h1b)

    def _lat_extra(h, extra_refs, out_refs):
        out_refs[0][...] = h[:, :32]
        out_refs[1][...] = h[:, 32:]
        z = jnp.dot(h[:, :32], extra_refs[0][...],
                    preferred_element_type=jnp.float32)
        out_refs[2][...] = z + jnp.dot(
            h[:, 32:], extra_refs[1][...], preferred_element_type=jnp.float32)

    lata, latb, z3 = _mid_layer(
        _lat_extra, [(NP_, 32), (NP_, 32), (NP_, 32)], h1a, h1b, p2a, p2b,
        dinv, Ws2[:32], Ws2[32:], Wn2[:32], Wn2[32:], b2.reshape(1, -1),
        extra_full=(Wv0n[:32], Wv0n[32:]))

    (p3,) = _segsum32x1(src, dst, zeros32, z3)
    g1, z4 = _layer3(lata, latb, p3, dinv, Wv0s[:32], Wv0s[32:],
                     bv0.reshape(1, -1), Wv1n)

    (p4,) = _segsum16x1(src, dst, zeros16, z4)
    g2 = _layer4(g1, p4, dinv, Wv1s, bv1.reshape(1, -1))

    (p5,) = _segsum16x1(src, dst, zeros16, g2)
    new_v = _layer5(g2, p5, dinv, vpad, Wv2s, Wv2n, bv2.reshape(1, -1))

    latent = jnp.concatenate([lata[:N], latb[:N]], axis=1)
    return (new_v[:N][None], latent[None])


# speedup vs baseline: 6.0799x; 1.1659x over previous
import functools

import jax
import jax.numpy as jnp
from jax import lax
from jax.experimental import pallas as pl
from jax.experimental.pallas import tpu as pltpu
from jax.experimental.pallas import tpu_sc as plsc

N = 50000
E = 800000
SKIP = 16
LAT = 64
G = 32

NP_ = 50176
BLK = 1024
NPB = NP_ // BLK
WORKERS = 32
PER_W = NP_ // WORKERS
PER_T = NP_ // 16
E_PAD = 819200
EC = E_PAD // WORKERS
CH = 320
NCH = EC // CH

_mesh = plsc.VectorSubcoreMesh(core_axis_name="c", subcore_axis_name="s",
                               num_cores=2, num_subcores=16)


PW8 = 8 * NP_ // WORKERS
CW = PW8 // 8


def _corner_gather_body(vol_ref, idx_ref, out_ref, iv0, iv1, r0, r1, s0, s1):
    c = lax.axis_index("c")
    s = lax.axis_index("s")
    wid = s * 2 + c
    base = wid * PW8
    pltpu.sync_copy(idx_ref.at[pl.ds(base, CW)], iv0)
    pltpu.make_async_copy(vol_ref.at[iv0], r0, s0).start()

    def it(i, carry):
        offa = base + (2 * i) * CW
        offb = base + (2 * i + 1) * CW
        pltpu.sync_copy(idx_ref.at[pl.ds(offb, CW)], iv1)
        pltpu.make_async_copy(vol_ref.at[iv1], r1, s1).start()
        pltpu.make_async_copy(vol_ref.at[iv0], r0, s0).wait()
        pltpu.sync_copy(r0, out_ref.at[pl.ds(offa, CW)])
        offc = base + lax.rem(2 * i + 2, 8) * CW
        pltpu.sync_copy(idx_ref.at[pl.ds(offc, CW)], iv0)
        pltpu.make_async_copy(vol_ref.at[iv0], r0, s0).start()
        pltpu.make_async_copy(vol_ref.at[iv1], r1, s1).wait()
        pltpu.sync_copy(r1, out_ref.at[pl.ds(offb, CW)])
        return carry

    lax.fori_loop(0, 4, it, 0)
    pltpu.make_async_copy(vol_ref.at[iv0], r0, s0).wait()


_corner_gather = pl.kernel(
    _corner_gather_body,
    out_type=jax.ShapeDtypeStruct((8 * NP_, SKIP), jnp.float32),
    mesh=_mesh,
    compiler_params=pltpu.CompilerParams(use_tc_tiling_on_sc=False),
    scratch_types=[
        pltpu.VMEM((CW,), jnp.int32),
        pltpu.VMEM((CW,), jnp.int32),
        pltpu.VMEM((CW, SKIP), jnp.float32),
        pltpu.VMEM((CW, SKIP), jnp.float32),
        pltpu.SemaphoreType.DMA,
        pltpu.SemaphoreType.DMA,
    ],
)


def _segsum_body(ntab, src_dst_zero_tabs_outs_scratch):
    refs = src_dst_zero_tabs_outs_scratch
    src_ref, dst_ref, zero_ref = refs[0], refs[1], refs[2]
    tabs = refs[3:3 + ntab]
    outs = refs[3 + ntab:3 + 2 * ntab]
    sv0, dv0, sv1, dv1, rows0, rows1, sem0, sem1, acc = refs[3 + 2 * ntab:]
    c = lax.axis_index("c")
    s = lax.axis_index("s")
    wid = s * 2 + c
    ebase = wid * EC
    rbase = s * PER_T
    obase = c * NP_ + s * PER_T
    for t in range(ntab):
        tab = tabs[t]
        pltpu.sync_copy(src_ref.at[pl.ds(ebase, CH)], sv0)
        pltpu.sync_copy(dst_ref.at[pl.ds(ebase, CH)], dv0)
        pltpu.make_async_copy(tab.at[sv0], rows0, sem0).start()
        pltpu.sync_copy(zero_ref.at[pl.ds(rbase, PER_T)],
                        acc.at[pl.ds(rbase, PER_T)])
        plsc.subcore_barrier()

        def it2(i, carry, _tab=tab):
            offb = ebase + (2 * i + 1) * CH
            pltpu.sync_copy(src_ref.at[pl.ds(offb, CH)], sv1)
            pltpu.sync_copy(dst_ref.at[pl.ds(offb, CH)], dv1)
            pltpu.make_async_copy(_tab.at[sv1], rows1, sem1).start()
            pltpu.make_async_copy(_tab.at[sv0], rows0, sem0).wait()
            pltpu.sync_copy(rows0, acc.at[dv0], add=True)
            offc = ebase + lax.rem(2 * i + 2, NCH) * CH
            pltpu.sync_copy(src_ref.at[pl.ds(offc, CH)], sv0)
            pltpu.sync_copy(dst_ref.at[pl.ds(offc, CH)], dv0)
            pltpu.make_async_copy(_tab.at[sv0], rows0, sem0).start()
            pltpu.make_async_copy(_tab.at[sv1], rows1, sem1).wait()
            pltpu.sync_copy(rows1, acc.at[dv1], add=True)
            return carry

        lax.fori_loop(0, NCH // 2, it2, 0)
        pltpu.make_async_copy(tab.at[sv0], rows0, sem0).wait()
        plsc.subcore_barrier()
        pltpu.sync_copy(acc.at[pl.ds(rbase, PER_T)],
                        outs[t].at[pl.ds(obase, PER_T)])


def _make_segsum(ntab, d):
    def body(*refs):
        _segsum_body(ntab, refs)

    return pl.kernel(
        body,
        out_type=[jax.ShapeDtypeStruct((2 * NP_, d), jnp.float32)] * ntab,
        mesh=_mesh,
        compiler_params=pltpu.CompilerParams(use_tc_tiling_on_sc=False),
        scratch_types=[
            pltpu.VMEM((CH,), jnp.int32),
            pltpu.VMEM((CH,), jnp.int32),
            pltpu.VMEM((CH,), jnp.int32),
            pltpu.VMEM((CH,), jnp.int32),
            pltpu.VMEM((CH, d), jnp.float32),
            pltpu.VMEM((CH, d), jnp.float32),
            pltpu.SemaphoreType.DMA,
            pltpu.SemaphoreType.DMA,
            pltpu.VMEM_SHARED((NP_, d), jnp.float32),
        ],
    )


_segsum32x1 = _make_segsum(1, 32)
_segsum32x2 = _make_segsum(2, 32)
_segsum16x1 = _make_segsum(1, 16)


def _row_spec(k):
    return pl.BlockSpec((BLK, k), lambda i: (i, 0))


def _p_specs(d):
    return [pl.BlockSpec((BLK, d), lambda i: (i, 0)),
            pl.BlockSpec((BLK, d), lambda i: (i + NPB, 0))]


def _full(a):
    return pl.BlockSpec(a.shape, lambda i: tuple(0 for _ in a.shape))


def _tri_prep_body(vt_ref, idx_ref, w_ref):
    half = 0.5 * (G - 1)
    x = (vt_ref[0:1, :] + 1.0) * half
    y = (vt_ref[1:2, :] + 1.0) * half
    z = (vt_ref[2:3, :] + 1.0) * half
    x0 = jnp.clip(jnp.floor(x), 0.0, G - 1)
    y0 = jnp.clip(jnp.floor(y), 0.0, G - 1)
    z0 = jnp.clip(jnp.floor(z), 0.0, G - 1)
    x1 = jnp.minimum(x0 + 1.0, G - 1)
    y1 = jnp.minimum(y0 + 1.0, G - 1)
    z1 = jnp.minimum(z0 + 1.0, G - 1)
    wx = x - x0
    wy = y - y0
    wz = z - z0
    x0i = x0.astype(jnp.int32); x1i = x1.astype(jnp.int32)
    y0i = y0.astype(jnp.int32); y1i = y1.astype(jnp.int32)
    z0i = z0.astype(jnp.int32); z1i = z1.astype(jnp.int32)

    def ind(zi, yi, xi):
        return (zi * G + yi) * G + xi

    idx_ref[...] = jnp.concatenate([
        ind(z0i, y0i, x0i), ind(z0i, y0i, x1i),
        ind(z0i, y1i, x0i), ind(z0i, y1i, x1i),
        ind(z1i, y0i, x0i), ind(z1i, y0i, x1i),
        ind(z1i, y1i, x0i), ind(z1i, y1i, x1i),
    ], axis=0)
    ax = 1.0 - wx
    ay = 1.0 - wy
    az = 1.0 - wz
    w_ref[...] = jnp.concatenate([
        az * ay * ax, az * ay * wx, az * wy * ax, az * wy * wx,
        wz * ay * ax, wz * ay * wx, wz * wy * ax, wz * wy * wx,
    ], axis=0)


def _tri_prep(vt):
    return pl.pallas_call(
        _tri_prep_body,
        grid=(NPB,),
        in_specs=[pl.BlockSpec((3, BLK), lambda i: (0, i))],
        out_specs=[pl.BlockSpec((8, BLK), lambda i: (0, i)),
                   pl.BlockSpec((8, BLK), lambda i: (0, i))],
        out_shape=[jax.ShapeDtypeStruct((8, NP_), jnp.int32),
                   jax.ShapeDtypeStruct((8, NP_), jnp.float32)],
    )(vt)


def _feats_body(r_ref, w_ref, v_ref, ws_ref, bs_ref, t0_ref):
    rows = r_ref[...]
    w = w_ref[...]
    sampled = jnp.zeros((BLK, SKIP), jnp.float32)
    for k in range(8):
        sampled = sampled + w[:, k:k + 1] * rows[k]
    skipped = jnp.dot(sampled, ws_ref[...],
                      preferred_element_type=jnp.float32) + bs_ref[...]
    ones = jnp.ones((BLK, 1), jnp.float32)
    zeros = jnp.zeros((BLK, 32 - (SKIP + 1) - 3 - 1), jnp.float32)
    t0_ref[...] = jnp.concatenate([skipped, v_ref[...], ones, zeros], axis=1)


def _feats(rows8, w8, vpad, W_skip, b_skip):
    return pl.pallas_call(
        _feats_body,
        grid=(NPB,),
        in_specs=[pl.BlockSpec((8, BLK, SKIP), lambda i: (0, i, 0)),
                  _row_spec(8), _row_spec(3), _full(W_skip), _full(b_skip)],
        out_specs=_row_spec(32),
        out_shape=jax.ShapeDtypeStruct((NP_, 32), jnp.float32),
    )(rows8, w8, vpad, W_skip, b_skip)


def _layer0_body(x_ref, pa_ref, pb_ref, ws_ref, wn_ref, b_ref,
                 ha_ref, hb_ref, dinv_ref):
    p = pa_ref[...] + pb_ref[...]
    deg = jnp.maximum(p[:, 20:21], 1.0)
    dinv = 1.0 / deg
    agg = p[:, :20] * dinv
    x = x_ref[...]
    h = jnp.dot(x[:, :20], ws_ref[...], preferred_element_type=jnp.float32)
    h = h + jnp.dot(agg, wn_ref[...], preferred_element_type=jnp.float32)
    h = jax.nn.relu(h + b_ref[...])
    ha_ref[...] = h[:, :32]
    hb_ref[...] = h[:, 32:]
    dinv_ref[...] = dinv


def _layer0(t0, p0, Ws0, Wn0, b0):
    return pl.pallas_call(
        _layer0_body,
        grid=(NPB,),
        in_specs=[_row_spec(32)] + _p_specs(32)
        + [_full(Ws0), _full(Wn0), _full(b0)],
        out_specs=[_row_spec(32), _row_spec(32), _row_spec(1)],
        out_shape=[jax.ShapeDtypeStruct((NP_, 32), jnp.float32),
                   jax.ShapeDtypeStruct((NP_, 32), jnp.float32),
                   jax.ShapeDtypeStruct((NP_, 1), jnp.float32)],
    )(t0, p0, p0, Ws0, Wn0, b0)


def _mid_body(n_extra, extra_fn, *refs):
    (xa_ref, xb_ref, paa_ref, pab_ref, pba_ref, pbb_ref, dinv_ref,
     wsa_ref, wsb_ref, wna_ref, wnb_ref, b_ref) = refs[:12]
    extra_refs = refs[12:12 + n_extra]
    out_refs = refs[12 + n_extra:]
    dinv = dinv_ref[...]
    agga = (paa_ref[...] + pab_ref[...]) * dinv
    aggb = (pba_ref[...] + pbb_ref[...]) * dinv
    h = jnp.dot(xa_ref[...], wsa_ref[...], preferred_element_type=jnp.float32)
    h = h + jnp.dot(xb_ref[...], wsb_ref[...],
                    preferred_element_type=jnp.float32)
    h = h + jnp.dot(agga, wna_ref[...], preferred_element_type=jnp.float32)
    h = h + jnp.dot(aggb, wnb_ref[...], preferred_element_type=jnp.float32)
    h = h + b_ref[...]
    extra_fn(h, extra_refs, out_refs)


def _relu_split(h, extra_refs, out_refs):
    h = jax.nn.relu(h)
    out_refs[0][...] = h[:, :32]
    out_refs[1][...] = h[:, 32:]


def _mid_layer(extra_fn, out_shapes, xa, xb, pa, pb, dinv,
               wsa, wsb, wna, wnb, b, extra_full=()):
    body = functools.partial(_mid_body, len(extra_full), extra_fn)
    return pl.pallas_call(
        body,
        grid=(NPB,),
        in_specs=[_row_spec(32), _row_spec(32)]
        + _p_specs(32) + _p_specs(32) + [_row_spec(1)]
        + [_full(wsa), _full(wsb), _full(wna), _full(wnb), _full(b)]
        + [_full(a) for a in extra_full],
        out_specs=[_row_spec(s[1]) for s in out_shapes],
        out_shape=[jax.ShapeDtypeStruct(s, jnp.float32) for s in out_shapes],
    )(xa, xb, pa, pa, pb, pb, dinv, wsa, wsb, wna, wnb, b, *extra_full)


def _layer3_body(la_ref, lb_ref, pa_ref, pb_ref, dinv_ref, wsa_ref, wsb_ref,
                 b_ref, wv1n_ref, g1_ref, z4_ref):
    agg = (pa_ref[...] + pb_ref[...]) * dinv_ref[...]
    h = jnp.dot(la_ref[...], wsa_ref[...], preferred_element_type=jnp.float32)
    h = h + jnp.dot(lb_ref[...], wsb_ref[...],
                    preferred_element_type=jnp.float32)
    g1 = jax.nn.relu(h + agg + b_ref[...])
    g1_ref[...] = g1
    z4_ref[...] = jnp.dot(g1, wv1n_ref[...],
                          preferred_element_type=jnp.float32)


def _layer3(la, lb, p3, dinv, wsa, wsb, bv0, Wv1n):
    return pl.pallas_call(
        _layer3_body,
        grid=(NPB,),
        in_specs=[_row_spec(32), _row_spec(32)] + _p_specs(32)
        + [_row_spec(1), _full(wsa), _full(wsb), _full(bv0), _full(Wv1n)],
        out_specs=[_row_spec(32), _row_spec(16)],
        out_shape=[jax.ShapeDtypeStruct((NP_, 32), jnp.float32),
                   jax.ShapeDtypeStruct((NP_, 16), jnp.float32)],
    )(la, lb, p3, p3, dinv, wsa, wsb, bv0, Wv1n)


def _layer4_body(g1_ref, pa_ref, pb_ref, dinv_ref, ws_ref, b_ref, g2_ref):
    agg = (pa_ref[...] + pb_ref[...]) * dinv_ref[...]
    h = jnp.dot(g1_ref[...], ws_ref[...], preferred_element_type=jnp.float32)
    g2_ref[...] = jax.nn.relu(h + agg + b_ref[...])


def _layer4(g1, p4, dinv, Wv1s, bv1):
    return pl.pallas_call(
        _layer4_body,
        grid=(NPB,),
        in_specs=[_row_spec(32)] + _p_specs(16)
        + [_row_spec(1), _full(Wv1s), _full(bv1)],
        out_specs=_row_spec(16),
        out_shape=jax.ShapeDtypeStruct((NP_, 16), jnp.float32),
    )(g1, p4, p4, dinv, Wv1s, bv1)


def _layer5_body(g2_ref, pa_ref, pb_ref, dinv_ref, v_ref, ws_ref, wn_ref,
                 b_ref, out_ref):
    agg = (pa_ref[...] + pb_ref[...]) * dinv_ref[...]
    d = jnp.dot(g2_ref[...], ws_ref[...], preferred_element_type=jnp.float32)
    d = d + jnp.dot(agg, wn_ref[...], preferred_element_type=jnp.float32)
    out_ref[...] = v_ref[...] + d + b_ref[...]


def _layer5(g2, p5, dinv, vpad, Wv2s, Wv2n, bv2):
    return pl.pallas_call(
        _layer5_body,
        grid=(NPB,),
        in_specs=[_row_spec(16)] + _p_specs(16)
        + [_row_spec(1), _row_spec(3), _full(Wv2s), _full(Wv2n), _full(bv2)],
        out_specs=_row_spec(3),
        out_shape=jax.ShapeDtypeStruct((NP_, 3), jnp.float32),
    )(g2, p5, p5, dinv, vpad, Wv2s, Wv2n, bv2)


def kernel(vertices, voxel_decoder_features, edge_index, W_skip, b_skip,
           Ws0, Wn0, b0, Ws1, Wn1, b1, Ws2, Wn2, b2, Wv0s, Wv0n, bv0,
           Wv1s, Wv1n, bv1, Wv2s, Wv2n, bv2):
    v = vertices[0]
    vpad = jnp.pad(v, ((0, NP_ - N), (0, 0)))
    vt = vpad.T
    vol = voxel_decoder_features[0, :SKIP].reshape(SKIP, G * G * G).T
    vol = jnp.asarray(vol, jnp.float32)
    src = jnp.pad(edge_index[0], (0, E_PAD - E))
    dst = jnp.pad(edge_index[1], (0, E_PAD - E), constant_values=N)
    zeros32 = jnp.zeros((NP_, 32), jnp.float32)
    zeros16 = jnp.zeros((NP_, 16), jnp.float32)

    idx8, w8r = _tri_prep(vt)
    rows8 = _corner_gather(vol, idx8.reshape(-1))
    t0 = _feats(rows8.reshape(8, NP_, SKIP), w8r.T, vpad, W_skip,
                b_skip.reshape(1, -1))

    (p0,) = _segsum32x1(src, dst, zeros32, t0)
    h0a, h0b, dinv = _layer0(t0, p0, Ws0[:20], Wn0[:20], b0.reshape(1, -1))

    p1a, p1b = _segsum32x2(src, dst, zeros32, h0a, h0b)
    h1a, h1b = _mid_layer(
        _relu_split, [(NP_, 32), (NP_, 32)], h0a, h0b, p1a, p1b, dinv,
        Ws1[:32], Ws1[32:], Wn1[:32], Wn1[32:], b1.reshape(1, -1))

    p2a, p2b = _segsum32x2(src, dst, zeros32, h1a, h1b)

    def _lat_extra(h, extra_refs, out_refs):
        out_refs[0][...] = h[:, :32]
        out_refs[1][...] = h[:, 32:]
        z = jnp.dot(h[:, :32], extra_refs[0][...],
                    preferred_element_type=jnp.float32)
        out_refs[2][...] = z + jnp.dot(
            h[:, 32:], extra_refs[1][...], preferred_element_type=jnp.float32)

    lata, latb, z3 = _mid_layer(
        _lat_extra, [(NP_, 32), (NP_, 32), (NP_, 32)], h1a, h1b, p2a, p2b,
        dinv, Ws2[:32], Ws2[32:], Wn2[:32], Wn2[32:], b2.reshape(1, -1),
        extra_full=(Wv0n[:32], Wv0n[32:]))

    (p3,) = _segsum32x1(src, dst, zeros32, z3)
    g1, z4 = _layer3(lata, latb, p3, dinv, Wv0s[:32], Wv0s[32:],
                     bv0.reshape(1, -1), Wv1n)

    (p4,) = _segsum16x1(src, dst, zeros16, z4)
    g2 = _layer4(g1, p4, dinv, Wv1s, bv1.reshape(1, -1))

    (p5,) = _segsum16x1(src, dst, zeros16, g2)
    new_v = _layer5(g2, p5, dinv, vpad, Wv2s, Wv2n, bv2.reshape(1, -1))

    latent = jnp.concatenate([lata[:N], latb[:N]], axis=1)
    return (new_v[:N][None], latent[None])


# speedup vs baseline: 6.1935x; 1.0187x over previous
import functools

import jax
import jax.numpy as jnp
from jax import lax
from jax.experimental import pallas as pl
from jax.experimental.pallas import tpu as pltpu
from jax.experimental.pallas import tpu_sc as plsc

N = 50000
E = 800000
SKIP = 16
LAT = 64
G = 32

NP_ = 50176
BLK = 1024
NPB = NP_ // BLK
WORKERS = 32
PER_W = NP_ // WORKERS
PER_T = NP_ // 16
E_PAD = 819200
EC = E_PAD // WORKERS
CH = 320
NCH = EC // CH
IB = 4
NBLK = NCH // IB

_mesh = plsc.VectorSubcoreMesh(core_axis_name="c", subcore_axis_name="s",
                               num_cores=2, num_subcores=16)


PW8 = 8 * NP_ // WORKERS
CW = PW8 // 8


def _corner_gather_body(vol_ref, idx_ref, out_ref, iv0, iv1, r0, r1, s0, s1):
    c = lax.axis_index("c")
    s = lax.axis_index("s")
    wid = s * 2 + c
    base = wid * PW8
    pltpu.sync_copy(idx_ref.at[pl.ds(base, CW)], iv0)
    pltpu.make_async_copy(vol_ref.at[iv0], r0, s0).start()

    def it(i, carry):
        offa = base + (2 * i) * CW
        offb = base + (2 * i + 1) * CW
        pltpu.sync_copy(idx_ref.at[pl.ds(offb, CW)], iv1)
        pltpu.make_async_copy(vol_ref.at[iv1], r1, s1).start()
        pltpu.make_async_copy(vol_ref.at[iv0], r0, s0).wait()
        pltpu.sync_copy(r0, out_ref.at[pl.ds(offa, CW)])
        offc = base + lax.rem(2 * i + 2, 8) * CW
        pltpu.sync_copy(idx_ref.at[pl.ds(offc, CW)], iv0)
        pltpu.make_async_copy(vol_ref.at[iv0], r0, s0).start()
        pltpu.make_async_copy(vol_ref.at[iv1], r1, s1).wait()
        pltpu.sync_copy(r1, out_ref.at[pl.ds(offb, CW)])
        return carry

    lax.fori_loop(0, 4, it, 0)
    pltpu.make_async_copy(vol_ref.at[iv0], r0, s0).wait()


_corner_gather = pl.kernel(
    _corner_gather_body,
    out_type=jax.ShapeDtypeStruct((8 * NP_, SKIP), jnp.float32),
    mesh=_mesh,
    compiler_params=pltpu.CompilerParams(use_tc_tiling_on_sc=False),
    scratch_types=[
        pltpu.VMEM((CW,), jnp.int32),
        pltpu.VMEM((CW,), jnp.int32),
        pltpu.VMEM((CW, SKIP), jnp.float32),
        pltpu.VMEM((CW, SKIP), jnp.float32),
        pltpu.SemaphoreType.DMA,
        pltpu.SemaphoreType.DMA,
    ],
)


def _segsum_body(ntab, src_dst_zero_tabs_outs_scratch):
    refs = src_dst_zero_tabs_outs_scratch
    src_ref, dst_ref, zero_ref = refs[0], refs[1], refs[2]
    tabs = refs[3:3 + ntab]
    outs = refs[3 + ntab:3 + 2 * ntab]
    (sb0, db0, sb1, db1, rows0, rows1, g0, g1, isem,
     acc) = refs[3 + 2 * ntab:]
    rows = (rows0, rows1)
    gsem = (g0, g1)
    c = lax.axis_index("c")
    s = lax.axis_index("s")
    wid = s * 2 + c
    wrow = wid * NCH
    rbase = s * PER_T
    obase = c * NP_ + s * PER_T

    def idx_rows(blk):
        return pl.ds(wrow + lax.rem(blk, NBLK) * IB, IB)

    def pf_start(blk, sb, db):
        pltpu.make_async_copy(src_ref.at[idx_rows(blk)], sb, isem).start()
        pltpu.make_async_copy(dst_ref.at[idx_rows(blk)], db, isem).start()

    def pf_wait(blk, sb, db):
        pltpu.make_async_copy(src_ref.at[idx_rows(blk)], sb, isem).wait()
        pltpu.make_async_copy(dst_ref.at[idx_rows(blk)], db, isem).wait()

    for t in range(ntab):
        tab = tabs[t]

        def g_start(sb, k, tab=tab):
            pltpu.make_async_copy(tab.at[sb.at[k]], rows[k & 1],
                                  gsem[k & 1]).start()

        def g_wait(sb, k, tab=tab):
            pltpu.make_async_copy(tab.at[sb.at[k]], rows[k & 1],
                                  gsem[k & 1]).wait()

        def sc_add(db, k):
            pltpu.sync_copy(rows[k & 1], acc.at[db.at[k]], add=True)

        pltpu.sync_copy(src_ref.at[idx_rows(0)], sb0)
        pltpu.sync_copy(dst_ref.at[idx_rows(0)], db0)
        g_start(sb0, 0)
        pf_start(1, sb1, db1)
        pltpu.sync_copy(zero_ref.at[pl.ds(rbase, PER_T)],
                        acc.at[pl.ds(rbase, PER_T)])
        plsc.subcore_barrier()

        def body(bp, carry):
            for k in range(1, IB):
                g_start(sb0, k)
                g_wait(sb0, k - 1)
                sc_add(db0, k - 1)
            pf_wait(2 * bp + 1, sb1, db1)
            g_start(sb1, 0)
            g_wait(sb0, IB - 1)
            sc_add(db0, IB - 1)
            pf_start(2 * bp + 2, sb0, db0)
            for k in range(1, IB):
                g_start(sb1, k)
                g_wait(sb1, k - 1)
                sc_add(db1, k - 1)
            pf_wait(2 * bp + 2, sb0, db0)
            g_start(sb0, 0)
            g_wait(sb1, IB - 1)
            sc_add(db1, IB - 1)
            pf_start(2 * bp + 3, sb1, db1)
            return carry

        lax.fori_loop(0, NBLK // 2, body, 0)
        g_wait(sb0, 0)
        pf_wait(NBLK + 1, sb1, db1)
        plsc.subcore_barrier()
        pltpu.sync_copy(acc.at[pl.ds(rbase, PER_T)],
                        outs[t].at[pl.ds(obase, PER_T)])


def _make_segsum(ntab, d):
    def body(*refs):
        _segsum_body(ntab, refs)

    return pl.kernel(
        body,
        out_type=[jax.ShapeDtypeStruct((2 * NP_, d), jnp.float32)] * ntab,
        mesh=_mesh,
        compiler_params=pltpu.CompilerParams(use_tc_tiling_on_sc=False),
        scratch_types=[
            pltpu.VMEM((IB, CH), jnp.int32),
            pltpu.VMEM((IB, CH), jnp.int32),
            pltpu.VMEM((IB, CH), jnp.int32),
            pltpu.VMEM((IB, CH), jnp.int32),
            pltpu.VMEM((CH, d), jnp.float32),
            pltpu.VMEM((CH, d), jnp.float32),
            pltpu.SemaphoreType.DMA,
            pltpu.SemaphoreType.DMA,
            pltpu.SemaphoreType.DMA,
            pltpu.VMEM_SHARED((NP_, d), jnp.float32),
        ],
    )


_segsum32x1 = _make_segsum(1, 32)
_segsum32x2 = _make_segsum(2, 32)
_segsum16x1 = _make_segsum(1, 16)


def _row_spec(k):
    return pl.BlockSpec((BLK, k), lambda i: (i, 0))


def _p_specs(d):
    return [pl.BlockSpec((BLK, d), lambda i: (i, 0)),
            pl.BlockSpec((BLK, d), lambda i: (i + NPB, 0))]


def _full(a):
    return pl.BlockSpec(a.shape, lambda i: tuple(0 for _ in a.shape))


def _tri_prep_body(vt_ref, idx_ref, w_ref):
    half = 0.5 * (G - 1)
    x = (vt_ref[0:1, :] + 1.0) * half
    y = (vt_ref[1:2, :] + 1.0) * half
    z = (vt_ref[2:3, :] + 1.0) * half
    x0 = jnp.clip(jnp.floor(x), 0.0, G - 1)
    y0 = jnp.clip(jnp.floor(y), 0.0, G - 1)
    z0 = jnp.clip(jnp.floor(z), 0.0, G - 1)
    x1 = jnp.minimum(x0 + 1.0, G - 1)
    y1 = jnp.minimum(y0 + 1.0, G - 1)
    z1 = jnp.minimum(z0 + 1.0, G - 1)
    wx = x - x0
    wy = y - y0
    wz = z - z0
    x0i = x0.astype(jnp.int32); x1i = x1.astype(jnp.int32)
    y0i = y0.astype(jnp.int32); y1i = y1.astype(jnp.int32)
    z0i = z0.astype(jnp.int32); z1i = z1.astype(jnp.int32)

    def ind(zi, yi, xi):
        return (zi * G + yi) * G + xi

    idx_ref[...] = jnp.concatenate([
        ind(z0i, y0i, x0i), ind(z0i, y0i, x1i),
        ind(z0i, y1i, x0i), ind(z0i, y1i, x1i),
        ind(z1i, y0i, x0i), ind(z1i, y0i, x1i),
        ind(z1i, y1i, x0i), ind(z1i, y1i, x1i),
    ], axis=0)
    ax = 1.0 - wx
    ay = 1.0 - wy
    az = 1.0 - wz
    w_ref[...] = jnp.concatenate([
        az * ay * ax, az * ay * wx, az * wy * ax, az * wy * wx,
        wz * ay * ax, wz * ay * wx, wz * wy * ax, wz * wy * wx,
    ], axis=0)


def _tri_prep(vt):
    return pl.pallas_call(
        _tri_prep_body,
        grid=(NPB,),
        in_specs=[pl.BlockSpec((3, BLK), lambda i: (0, i))],
        out_specs=[pl.BlockSpec((8, BLK), lambda i: (0, i)),
                   pl.BlockSpec((8, BLK), lambda i: (0, i))],
        out_shape=[jax.ShapeDtypeStruct((8, NP_), jnp.int32),
                   jax.ShapeDtypeStruct((8, NP_), jnp.float32)],
    )(vt)


def _feats_body(r_ref, w_ref, v_ref, ws_ref, bs_ref, t0_ref):
    rows = r_ref[...]
    w = w_ref[...]
    sampled = jnp.zeros((BLK, SKIP), jnp.float32)
    for k in range(8):
        sampled = sampled + w[:, k:k + 1] * rows[k]
    skipped = jnp.dot(sampled, ws_ref[...],
                      preferred_element_type=jnp.float32) + bs_ref[...]
    ones = jnp.ones((BLK, 1), jnp.float32)
    zeros = jnp.zeros((BLK, 32 - (SKIP + 1) - 3 - 1), jnp.float32)
    t0_ref[...] = jnp.concatenate([skipped, v_ref[...], ones, zeros], axis=1)


def _feats(rows8, w8, vpad, W_skip, b_skip):
    return pl.pallas_call(
        _feats_body,
        grid=(NPB,),
        in_specs=[pl.BlockSpec((8, BLK, SKIP), lambda i: (0, i, 0)),
                  _row_spec(8), _row_spec(3), _full(W_skip), _full(b_skip)],
        out_specs=_row_spec(32),
        out_shape=jax.ShapeDtypeStruct((NP_, 32), jnp.float32),
    )(rows8, w8, vpad, W_skip, b_skip)


def _layer0_body(x_ref, pa_ref, pb_ref, ws_ref, wn_ref, b_ref,
                 ha_ref, hb_ref, dinv_ref):
    p = pa_ref[...] + pb_ref[...]
    deg = jnp.maximum(p[:, 20:21], 1.0)
    dinv = 1.0 / deg
    agg = p[:, :20] * dinv
    x = x_ref[...]
    h = jnp.dot(x[:, :20], ws_ref[...], preferred_element_type=jnp.float32)
    h = h + jnp.dot(agg, wn_ref[...], preferred_element_type=jnp.float32)
    h = jax.nn.relu(h + b_ref[...])
    ha_ref[...] = h[:, :32]
    hb_ref[...] = h[:, 32:]
    dinv_ref[...] = dinv


def _layer0(t0, p0, Ws0, Wn0, b0):
    return pl.pallas_call(
        _layer0_body,
        grid=(NPB,),
        in_specs=[_row_spec(32)] + _p_specs(32)
        + [_full(Ws0), _full(Wn0), _full(b0)],
        out_specs=[_row_spec(32), _row_spec(32), _row_spec(1)],
        out_shape=[jax.ShapeDtypeStruct((NP_, 32), jnp.float32),
                   jax.ShapeDtypeStruct((NP_, 32), jnp.float32),
                   jax.ShapeDtypeStruct((NP_, 1), jnp.float32)],
    )(t0, p0, p0, Ws0, Wn0, b0)


def _mid_body(n_extra, extra_fn, *refs):
    (xa_ref, xb_ref, paa_ref, pab_ref, pba_ref, pbb_ref, dinv_ref,
     wsa_ref, wsb_ref, wna_ref, wnb_ref, b_ref) = refs[:12]
    extra_refs = refs[12:12 + n_extra]
    out_refs = refs[12 + n_extra:]
    dinv = dinv_ref[...]
    agga = (paa_ref[...] + pab_ref[...]) * dinv
    aggb = (pba_ref[...] + pbb_ref[...]) * dinv
    h = jnp.dot(xa_ref[...], wsa_ref[...], preferred_element_type=jnp.float32)
    h = h + jnp.dot(xb_ref[...], wsb_ref[...],
                    preferred_element_type=jnp.float32)
    h = h + jnp.dot(agga, wna_ref[...], preferred_element_type=jnp.float32)
    h = h + jnp.dot(aggb, wnb_ref[...], preferred_element_type=jnp.float32)
    h = h + b_ref[...]
    extra_fn(h, extra_refs, out_refs)


def _relu_split(h, extra_refs, out_refs):
    h = jax.nn.relu(h)
    out_refs[0][...] = h[:, :32]
    out_refs[1][...] = h[:, 32:]


def _mid_layer(extra_fn, out_shapes, xa, xb, pa, pb, dinv,
               wsa, wsb, wna, wnb, b, extra_full=()):
    body = functools.partial(_mid_body, len(extra_full), extra_fn)
    return pl.pallas_call(
        body,
        grid=(NPB,),
        in_specs=[_row_spec(32), _row_spec(32)]
        + _p_specs(32) + _p_specs(32) + [_row_spec(1)]
        + [_full(wsa), _full(wsb), _full(wna), _full(wnb), _full(b)]
        + [_full(a) for a in extra_full],
        out_specs=[_row_spec(s[1]) for s in out_shapes],
        out_shape=[jax.ShapeDtypeStruct(s, jnp.float32) for s in out_shapes],
    )(xa, xb, pa, pa, pb, pb, dinv, wsa, wsb, wna, wnb, b, *extra_full)


def _layer3_body(la_ref, lb_ref, pa_ref, pb_ref, dinv_ref, wsa_ref, wsb_ref,
                 b_ref, wv1n_ref, g1_ref, z4_ref):
    agg = (pa_ref[...] + pb_ref[...]) * dinv_ref[...]
    h = jnp.dot(la_ref[...], wsa_ref[...], preferred_element_type=jnp.float32)
    h = h + jnp.dot(lb_ref[...], wsb_ref[...],
                    preferred_element_type=jnp.float32)
    g1 = jax.nn.relu(h + agg + b_ref[...])
    g1_ref[...] = g1
    z4_ref[...] = jnp.dot(g1, wv1n_ref[...],
                          preferred_element_type=jnp.float32)


def _layer3(la, lb, p3, dinv, wsa, wsb, bv0, Wv1n):
    return pl.pallas_call(
        _layer3_body,
        grid=(NPB,),
        in_specs=[_row_spec(32), _row_spec(32)] + _p_specs(32)
        + [_row_spec(1), _full(wsa), _full(wsb), _full(bv0), _full(Wv1n)],
        out_specs=[_row_spec(32), _row_spec(16)],
        out_shape=[jax.ShapeDtypeStruct((NP_, 32), jnp.float32),
                   jax.ShapeDtypeStruct((NP_, 16), jnp.float32)],
    )(la, lb, p3, p3, dinv, wsa, wsb, bv0, Wv1n)


def _layer4_body(g1_ref, pa_ref, pb_ref, dinv_ref, ws_ref, b_ref, g2_ref):
    agg = (pa_ref[...] + pb_ref[...]) * dinv_ref[...]
    h = jnp.dot(g1_ref[...], ws_ref[...], preferred_element_type=jnp.float32)
    g2_ref[...] = jax.nn.relu(h + agg + b_ref[...])


def _layer4(g1, p4, dinv, Wv1s, bv1):
    return pl.pallas_call(
        _layer4_body,
        grid=(NPB,),
        in_specs=[_row_spec(32)] + _p_specs(16)
        + [_row_spec(1), _full(Wv1s), _full(bv1)],
        out_specs=_row_spec(16),
        out_shape=jax.ShapeDtypeStruct((NP_, 16), jnp.float32),
    )(g1, p4, p4, dinv, Wv1s, bv1)


def _layer5_body(g2_ref, pa_ref, pb_ref, dinv_ref, v_ref, ws_ref, wn_ref,
                 b_ref, out_ref):
    agg = (pa_ref[...] + pb_ref[...]) * dinv_ref[...]
    d = jnp.dot(g2_ref[...], ws_ref[...], preferred_element_type=jnp.float32)
    d = d + jnp.dot(agg, wn_ref[...], preferred_element_type=jnp.float32)
    out_ref[...] = v_ref[...] + d + b_ref[...]


def _layer5(g2, p5, dinv, vpad, Wv2s, Wv2n, bv2):
    return pl.pallas_call(
        _layer5_body,
        grid=(NPB,),
        in_specs=[_row_spec(16)] + _p_specs(16)
        + [_row_spec(1), _row_spec(3), _full(Wv2s), _full(Wv2n), _full(bv2)],
        out_specs=_row_spec(3),
        out_shape=jax.ShapeDtypeStruct((NP_, 3), jnp.float32),
    )(g2, p5, p5, dinv, vpad, Wv2s, Wv2n, bv2)


def kernel(vertices, voxel_decoder_features, edge_index, W_skip, b_skip,
           Ws0, Wn0, b0, Ws1, Wn1, b1, Ws2, Wn2, b2, Wv0s, Wv0n, bv0,
           Wv1s, Wv1n, bv1, Wv2s, Wv2n, bv2):
    v = vertices[0]
    vpad = jnp.pad(v, ((0, NP_ - N), (0, 0)))
    vt = vpad.T
    vol = voxel_decoder_features[0, :SKIP].reshape(SKIP, G * G * G).T
    vol = jnp.asarray(vol, jnp.float32)
    src = jnp.pad(edge_index[0], (0, E_PAD - E)).reshape(-1, CH)
    dst = jnp.pad(edge_index[1], (0, E_PAD - E),
                  constant_values=N).reshape(-1, CH)
    zeros32 = jnp.zeros((NP_, 32), jnp.float32)
    zeros16 = jnp.zeros((NP_, 16), jnp.float32)

    idx8, w8r = _tri_prep(vt)
    rows8 = _corner_gather(vol, idx8.reshape(-1))
    t0 = _feats(rows8.reshape(8, NP_, SKIP), w8r.T, vpad, W_skip,
                b_skip.reshape(1, -1))

    (p0,) = _segsum32x1(src, dst, zeros32, t0)
    h0a, h0b, dinv = _layer0(t0, p0, Ws0[:20], Wn0[:20], b0.reshape(1, -1))

    p1a, p1b = _segsum32x2(src, dst, zeros32, h0a, h0b)
    h1a, h1b = _mid_layer(
        _relu_split, [(NP_, 32), (NP_, 32)], h0a, h0b, p1a, p1b, dinv,
        Ws1[:32], Ws1[32:], Wn1[:32], Wn1[32:], b1.reshape(1, -1))

    p2a, p2b = _segsum32x2(src, dst, zeros32, h1a, h1b)

    def _lat_extra(h, extra_refs, out_refs):
        out_refs[0][...] = h[:, :32]
        out_refs[1][...] = h[:, 32:]
        z = jnp.dot(h[:, :32], extra_refs[0][...],
                    preferred_element_type=jnp.float32)
        out_refs[2][...] = z + jnp.dot(
            h[:, 32:], extra_refs[1][...], preferred_element_type=jnp.float32)

    lata, latb, z3 = _mid_layer(
        _lat_extra, [(NP_, 32), (NP_, 32), (NP_, 32)], h1a, h1b, p2a, p2b,
        dinv, Ws2[:32], Ws2[32:], Wn2[:32], Wn2[32:], b2.reshape(1, -1),
        extra_full=(Wv0n[:32], Wv0n[32:]))

    (p3,) = _segsum32x1(src, dst, zeros32, z3)
    g1, z4 = _layer3(lata, latb, p3, dinv, Wv0s[:32], Wv0s[32:],
                     bv0.reshape(1, -1), Wv1n)

    (p4,) = _segsum16x1(src, dst, zeros16, z4)
    g2 = _layer4(g1, p4, dinv, Wv1s, bv1.reshape(1, -1))

    (p5,) = _segsum16x1(src, dst, zeros16, g2)
    new_v = _layer5(g2, p5, dinv, vpad, Wv2s, Wv2n, bv2.reshape(1, -1))

    latent = jnp.concatenate([lata[:N], latb[:N]], axis=1)
    return (new_v[:N][None], latent[None])
